# Initial kernel scaffold; baseline (speedup 1.0000x reference)
import jax
import jax.numpy as jnp
from jax.experimental import pallas as pl


def kernel(x, edge_index):
    raise NotImplementedError("write your pallas kernel here")


# speedup vs baseline: 8.5736x; 8.5736x over previous
import functools

import jax
import jax.numpy as jnp
from jax import lax
from jax.experimental import pallas as pl
from jax.experimental.pallas import tpu as pltpu
from jax.experimental.pallas import tpu_sc as plsc

N = 10000
E = 320000
D = 128
NC, NS = 2, 16
NW = NC * NS
K = 128
CHUNKS = 80
EPT = CHUNKS * K
EPAD = NW * EPT
NPAD = 10240
RPT = NPAD // NS
SROWS = 80
NSTEP = RPT // SROWS

_mesh = plsc.VectorSubcoreMesh(
    core_axis_name="c", subcore_axis_name="s", num_cores=NC, num_subcores=NS)


def _deg_body(dst_hbm, ones_hbm, zeros_hbm, out_hbm,
              di_v, ones_v, slab_v, deg_sh):
    c = lax.axis_index("c")
    s = lax.axis_index("s")
    wid = s * NC + c
    row0 = s * RPT

    pltpu.sync_copy(ones_hbm, ones_v)

    pltpu.sync_copy(zeros_hbm, slab_v)

    def _zstep(j, _):
        pltpu.sync_copy(slab_v, deg_sh.at[pl.ds(row0 + j * SROWS, SROWS)])
        return 0
    lax.fori_loop(0, RPT // SROWS, _zstep, 0)
    plsc.subcore_barrier()

    def _step(i, _):
        pltpu.sync_copy(dst_hbm.at[pl.ds((wid * CHUNKS + i) * K, K)], di_v)
        pltpu.sync_copy(ones_v, deg_sh.at[di_v], add=True)
        return 0
    lax.fori_loop(0, CHUNKS, _step, 0)

    plsc.subcore_barrier()

    def _dstep(j, _):
        pltpu.sync_copy(deg_sh.at[pl.ds(row0 + j * SROWS, SROWS)], slab_v)
        pltpu.sync_copy(
            slab_v, out_hbm.at[pl.ds(c * NPAD + row0 + j * SROWS, SROWS)])
        return 0
    lax.fori_loop(0, RPT // SROWS, _dstep, 0)


_deg_call = functools.partial(
    pl.kernel,
    out_type=jax.ShapeDtypeStruct((NC * NPAD, D), jnp.float32),
    mesh=_mesh,
    scratch_types=[
        pltpu.VMEM((K,), jnp.int32),
        pltpu.VMEM((K, D), jnp.float32),
        pltpu.VMEM((SROWS, D), jnp.float32),
        pltpu.VMEM_SHARED((NPAD, D), jnp.float32),
    ],
)(_deg_body)


def _scat_body(xp_hbm, src_hbm, dst_hbm, zeros_hbm, out_hbm,
               si_v, di_v, rows_v, slab_v, acc_sh, sem):
    c = lax.axis_index("c")
    s = lax.axis_index("s")
    wid = s * NC + c
    row0 = s * RPT

    pltpu.sync_copy(zeros_hbm, slab_v)

    def _zstep(j, _):
        pltpu.sync_copy(slab_v, acc_sh.at[pl.ds(row0 + j * SROWS, SROWS)])
        return 0
    lax.fori_loop(0, NSTEP, _zstep, 0)
    plsc.subcore_barrier()

    def _step(i, _):
        eoff = (wid * CHUNKS + i) * K
        pltpu.sync_copy(src_hbm.at[pl.ds(eoff, K)], si_v)
        pltpu.sync_copy(dst_hbm.at[pl.ds(eoff, K)], di_v)
        pltpu.async_copy(xp_hbm.at[si_v], rows_v, sem).wait()
        pltpu.sync_copy(rows_v, acc_sh.at[di_v], add=True)
        return 0
    lax.fori_loop(0, CHUNKS, _step, 0)

    plsc.subcore_barrier()

    def _dstep(j, _):
        pltpu.sync_copy(acc_sh.at[pl.ds(row0 + j * SROWS, SROWS)], slab_v)
        pltpu.sync_copy(
            slab_v, out_hbm.at[pl.ds(c * NPAD + row0 + j * SROWS, SROWS)])
        return 0
    lax.fori_loop(0, NSTEP, _dstep, 0)


_scat_call = functools.partial(
    pl.kernel,
    out_type=jax.ShapeDtypeStruct((NC * NPAD, D), jnp.float32),
    mesh=_mesh,
    scratch_types=[
        pltpu.VMEM((K,), jnp.int32),
        pltpu.VMEM((K,), jnp.int32),
        pltpu.VMEM((K, D), jnp.float32),
        pltpu.VMEM((SROWS, D), jnp.float32),
        pltpu.VMEM_SHARED((NPAD, D), jnp.float32),
        pltpu.SemaphoreType.DMA,
    ],
)(_scat_body)


def _pre_body(dp_ref, x_ref, xp_ref, dinv_ref):
    deg = dp_ref[0:NPAD, 0:1] + dp_ref[NPAD:2 * NPAD, 0:1]
    pos = deg > 0.0
    dinv = jnp.where(pos, lax.rsqrt(jnp.where(pos, deg, 1.0)), 0.0)
    dinv_ref[...] = dinv
    xp_ref[...] = x_ref[...] * dinv


_pre_call = pl.pallas_call(
    _pre_body,
    out_shape=(jax.ShapeDtypeStruct((NPAD, D), jnp.float32),
               jax.ShapeDtypeStruct((NPAD, 1), jnp.float32)),
)


def _post_body(acc_ref, dinv_ref, o_ref):
    o_ref[...] = (acc_ref[0:N, :] + acc_ref[NPAD:NPAD + N, :]) * dinv_ref[0:N]


_post_call = pl.pallas_call(
    _post_body,
    out_shape=jax.ShapeDtypeStruct((N, D), jnp.float32),
)


def kernel(x, edge_index):
    src = edge_index[0].astype(jnp.int32)
    dst = edge_index[1].astype(jnp.int32)
    padidx = jnp.full((EPAD - E,), N, jnp.int32)
    src_p = jnp.concatenate([src, padidx])
    dst_p = jnp.concatenate([dst, padidx])
    x_pad = jnp.pad(x, ((0, NPAD - N), (0, 0)))

    ones_d = jnp.ones((K, D), jnp.float32)
    zeros_d = jnp.zeros((SROWS, D), jnp.float32)

    dp = _deg_call(dst_p, ones_d, zeros_d)
    xp, dinv = _pre_call(dp, x_pad)
    acc = _scat_call(xp, src_p, dst_p, zeros_d)
    return _post_call(acc, dinv)


# speedup vs baseline: 10.1919x; 1.1888x over previous
import functools

import jax
import jax.numpy as jnp
from jax import lax
from jax.experimental import pallas as pl
from jax.experimental.pallas import tpu as pltpu
from jax.experimental.pallas import tpu_sc as plsc

N = 10000
E = 320000
D = 128
NC, NS = 2, 16
NW = NC * NS
K = 128
CHUNKS = 80
EPT = CHUNKS * K
EPAD = NW * EPT
NPAD = 10240
RPT = NPAD // NS
SROWS = 80
NSTEP = RPT // SROWS

_mesh = plsc.VectorSubcoreMesh(
    core_axis_name="c", subcore_axis_name="s", num_cores=NC, num_subcores=NS)


def _deg_body(dst_hbm, ones_hbm, zeros_hbm, out_hbm,
              di_v, ones_v, slab_v, deg_sh):
    c = lax.axis_index("c")
    s = lax.axis_index("s")
    wid = s * NC + c
    row0 = s * RPT

    pltpu.sync_copy(ones_hbm, ones_v)

    pltpu.sync_copy(zeros_hbm, slab_v)

    def _zstep(j, _):
        pltpu.sync_copy(slab_v, deg_sh.at[pl.ds(row0 + j * SROWS, SROWS)])
        return 0
    lax.fori_loop(0, RPT // SROWS, _zstep, 0)
    plsc.subcore_barrier()

    def _step(i, _):
        pltpu.sync_copy(dst_hbm.at[pl.ds((wid * CHUNKS + i) * K, K)], di_v)
        pltpu.sync_copy(ones_v, deg_sh.at[di_v], add=True)
        return 0
    lax.fori_loop(0, CHUNKS, _step, 0)

    plsc.subcore_barrier()

    def _dstep(j, _):
        pltpu.sync_copy(deg_sh.at[pl.ds(row0 + j * SROWS, SROWS)], slab_v)
        pltpu.sync_copy(
            slab_v, out_hbm.at[pl.ds(c * NPAD + row0 + j * SROWS, SROWS)])
        return 0
    lax.fori_loop(0, RPT // SROWS, _dstep, 0)


_deg_call = functools.partial(
    pl.kernel,
    out_type=jax.ShapeDtypeStruct((NC * NPAD, D), jnp.float32),
    mesh=_mesh,
    scratch_types=[
        pltpu.VMEM((K,), jnp.int32),
        pltpu.VMEM((K, D), jnp.float32),
        pltpu.VMEM((SROWS, D), jnp.float32),
        pltpu.VMEM_SHARED((NPAD, D), jnp.float32),
    ],
)(_deg_body)


def _scat_body(xp_hbm, src_hbm, dst_hbm, zeros_hbm, out_hbm,
               si0, di0, si1, di1, rows0, rows1, slab_v, acc_sh,
               gsem0, gsem1):
    c = lax.axis_index("c")
    s = lax.axis_index("s")
    wid = s * NC + c
    row0 = s * RPT

    pltpu.sync_copy(zeros_hbm, slab_v)

    def _zstep(j, _):
        pltpu.sync_copy(slab_v, acc_sh.at[pl.ds(row0 + j * SROWS, SROWS)])
        return 0
    lax.fori_loop(0, NSTEP, _zstep, 0)
    plsc.subcore_barrier()

    base = wid * CHUNKS

    def _stage(k, si, di):
        eoff = (base + k) * K
        pltpu.sync_copy(src_hbm.at[pl.ds(eoff, K)], si)
        pltpu.sync_copy(dst_hbm.at[pl.ds(eoff, K)], di)

    _stage(0, si0, di0)
    pltpu.async_copy(xp_hbm.at[si0], rows0, gsem0)

    TPAIR = CHUNKS // 2

    def _pair(t, _):
        k0 = 2 * t
        _stage(k0 + 1, si1, di1)
        pltpu.async_copy(xp_hbm.at[si1], rows1, gsem1)
        pltpu.make_async_copy(xp_hbm.at[si0], rows0, gsem0).wait()
        pltpu.sync_copy(rows0, acc_sh.at[di0], add=True)

        @pl.when(t < TPAIR - 1)
        def _():
            _stage(k0 + 2, si0, di0)
            pltpu.async_copy(xp_hbm.at[si0], rows0, gsem0)
        pltpu.make_async_copy(xp_hbm.at[si1], rows1, gsem1).wait()
        pltpu.sync_copy(rows1, acc_sh.at[di1], add=True)
        return 0
    lax.fori_loop(0, TPAIR, _pair, 0)

    plsc.subcore_barrier()

    def _dstep(j, _):
        pltpu.sync_copy(acc_sh.at[pl.ds(row0 + j * SROWS, SROWS)], slab_v)
        pltpu.sync_copy(
            slab_v, out_hbm.at[pl.ds(c * NPAD + row0 + j * SROWS, SROWS)])
        return 0
    lax.fori_loop(0, NSTEP, _dstep, 0)


_scat_call = functools.partial(
    pl.kernel,
    out_type=jax.ShapeDtypeStruct((NC * NPAD, D), jnp.float32),
    mesh=_mesh,
    scratch_types=[
        pltpu.VMEM((K,), jnp.int32),
        pltpu.VMEM((K,), jnp.int32),
        pltpu.VMEM((K,), jnp.int32),
        pltpu.VMEM((K,), jnp.int32),
        pltpu.VMEM((K, D), jnp.float32),
        pltpu.VMEM((K, D), jnp.float32),
        pltpu.VMEM((SROWS, D), jnp.float32),
        pltpu.VMEM_SHARED((NPAD, D), jnp.float32),
        pltpu.SemaphoreType.DMA,
        pltpu.SemaphoreType.DMA,
    ],
)(_scat_body)


def _pre_body(dp_ref, x_ref, xp_ref, dinv_ref):
    deg = dp_ref[0:NPAD, 0:1] + dp_ref[NPAD:2 * NPAD, 0:1]
    pos = deg > 0.0
    dinv = jnp.where(pos, lax.rsqrt(jnp.where(pos, deg, 1.0)), 0.0)
    dinv_ref[...] = dinv
    xp_ref[...] = x_ref[...] * dinv


_pre_call = pl.pallas_call(
    _pre_body,
    out_shape=(jax.ShapeDtypeStruct((NPAD, D), jnp.float32),
               jax.ShapeDtypeStruct((NPAD, 1), jnp.float32)),
)


def _post_body(acc_ref, dinv_ref, o_ref):
    o_ref[...] = (acc_ref[0:N, :] + acc_ref[NPAD:NPAD + N, :]) * dinv_ref[0:N]


_post_call = pl.pallas_call(
    _post_body,
    out_shape=jax.ShapeDtypeStruct((N, D), jnp.float32),
)


def kernel(x, edge_index):
    src = edge_index[0].astype(jnp.int32)
    dst = edge_index[1].astype(jnp.int32)
    padidx = jnp.full((EPAD - E,), N, jnp.int32)
    src_p = jnp.concatenate([src, padidx])
    dst_p = jnp.concatenate([dst, padidx])
    x_pad = jnp.pad(x, ((0, NPAD - N), (0, 0)))

    ones_d = jnp.ones((K, D), jnp.float32)
    zeros_d = jnp.zeros((SROWS, D), jnp.float32)

    dp = _deg_call(dst_p, ones_d, zeros_d)
    xp, dinv = _pre_call(dp, x_pad)
    acc = _scat_call(xp, src_p, dst_p, zeros_d)
    return _post_call(acc, dinv)


# speedup vs baseline: 10.7438x; 1.0541x over previous
import functools

import jax
import jax.numpy as jnp
from jax import lax
from jax.experimental import pallas as pl
from jax.experimental.pallas import tpu as pltpu
from jax.experimental.pallas import tpu_sc as plsc

N = 10000
E = 320000
D = 128
NC, NS = 2, 16
NW = NC * NS
K = 128
CHUNKS = 80
EPT = CHUNKS * K
EPAD = NW * EPT
NPAD = 10240
RPT = NPAD // NS
SROWS = 80
NSTEP = RPT // SROWS

_mesh = plsc.VectorSubcoreMesh(
    core_axis_name="c", subcore_axis_name="s", num_cores=NC, num_subcores=NS)


def _deg_body(dst_hbm, ones_hbm, zeros_hbm, out_hbm,
              di0, di1, ones_v, slab_v, deg_sh, isem0, isem1):
    c = lax.axis_index("c")
    s = lax.axis_index("s")
    wid = s * NC + c
    row0 = s * RPT

    pltpu.sync_copy(ones_hbm, ones_v)

    pltpu.sync_copy(zeros_hbm, slab_v)

    def _zstep(j, _):
        pltpu.sync_copy(slab_v, deg_sh.at[pl.ds(row0 + j * SROWS, SROWS)])
        return 0
    lax.fori_loop(0, RPT // SROWS, _zstep, 0)
    plsc.subcore_barrier()

    base = wid * CHUNKS
    pltpu.async_copy(dst_hbm.at[pl.ds(base * K, K)], di0, isem0)

    def _pair(t, _):
        k0 = 2 * t
        pltpu.async_copy(dst_hbm.at[pl.ds((base + k0 + 1) * K, K)], di1,
                         isem1)
        pltpu.make_async_copy(
            dst_hbm.at[pl.ds((base + k0) * K, K)], di0, isem0).wait()
        pltpu.sync_copy(ones_v, deg_sh.at[di0], add=True)

        @pl.when(t < CHUNKS // 2 - 1)
        def _():
            pltpu.async_copy(dst_hbm.at[pl.ds((base + k0 + 2) * K, K)], di0,
                             isem0)
        pltpu.make_async_copy(
            dst_hbm.at[pl.ds((base + k0 + 1) * K, K)], di1, isem1).wait()
        pltpu.sync_copy(ones_v, deg_sh.at[di1], add=True)
        return 0
    lax.fori_loop(0, CHUNKS // 2, _pair, 0)

    plsc.subcore_barrier()

    def _dstep(j, _):
        pltpu.sync_copy(deg_sh.at[pl.ds(row0 + j * SROWS, SROWS)], slab_v)
        pltpu.sync_copy(
            slab_v, out_hbm.at[pl.ds(c * NPAD + row0 + j * SROWS, SROWS)])
        return 0
    lax.fori_loop(0, RPT // SROWS, _dstep, 0)


_deg_call = functools.partial(
    pl.kernel,
    out_type=jax.ShapeDtypeStruct((NC * NPAD, D), jnp.float32),
    mesh=_mesh,
    scratch_types=[
        pltpu.VMEM((K,), jnp.int32),
        pltpu.VMEM((K,), jnp.int32),
        pltpu.VMEM((K, D), jnp.float32),
        pltpu.VMEM((SROWS, D), jnp.float32),
        pltpu.VMEM_SHARED((NPAD, D), jnp.float32),
        pltpu.SemaphoreType.DMA,
        pltpu.SemaphoreType.DMA,
    ],
)(_deg_body)


def _scat_body(xp_hbm, src_hbm, dst_hbm, zeros_hbm, out_hbm,
               si0, di0, si1, di1, rows0, rows1, slab_v, acc_sh,
               gsem0, gsem1):
    c = lax.axis_index("c")
    s = lax.axis_index("s")
    wid = s * NC + c
    row0 = s * RPT

    pltpu.sync_copy(zeros_hbm, slab_v)

    def _zstep(j, _):
        pltpu.sync_copy(slab_v, acc_sh.at[pl.ds(row0 + j * SROWS, SROWS)])
        return 0
    lax.fori_loop(0, NSTEP, _zstep, 0)
    plsc.subcore_barrier()

    base = wid * CHUNKS

    def _stage(k, si, di):
        eoff = (base + k) * K
        pltpu.sync_copy(src_hbm.at[pl.ds(eoff, K)], si)
        pltpu.sync_copy(dst_hbm.at[pl.ds(eoff, K)], di)

    _stage(0, si0, di0)
    pltpu.async_copy(xp_hbm.at[si0], rows0, gsem0)

    TPAIR = CHUNKS // 2

    def _pair(t, _):
        k0 = 2 * t
        _stage(k0 + 1, si1, di1)
        pltpu.async_copy(xp_hbm.at[si1], rows1, gsem1)
        pltpu.make_async_copy(xp_hbm.at[si0], rows0, gsem0).wait()
        pltpu.sync_copy(rows0, acc_sh.at[di0], add=True)

        @pl.when(t < TPAIR - 1)
        def _():
            _stage(k0 + 2, si0, di0)
            pltpu.async_copy(xp_hbm.at[si0], rows0, gsem0)
        pltpu.make_async_copy(xp_hbm.at[si1], rows1, gsem1).wait()
        pltpu.sync_copy(rows1, acc_sh.at[di1], add=True)
        return 0
    lax.fori_loop(0, TPAIR, _pair, 0)

    plsc.subcore_barrier()

    def _dstep(j, _):
        pltpu.sync_copy(acc_sh.at[pl.ds(row0 + j * SROWS, SROWS)], slab_v)
        pltpu.sync_copy(
            slab_v, out_hbm.at[pl.ds(c * NPAD + row0 + j * SROWS, SROWS)])
        return 0
    lax.fori_loop(0, NSTEP, _dstep, 0)


_scat_call = functools.partial(
    pl.kernel,
    out_type=jax.ShapeDtypeStruct((NC * NPAD, D), jnp.float32),
    mesh=_mesh,
    scratch_types=[
        pltpu.VMEM((K,), jnp.int32),
        pltpu.VMEM((K,), jnp.int32),
        pltpu.VMEM((K,), jnp.int32),
        pltpu.VMEM((K,), jnp.int32),
        pltpu.VMEM((K, D), jnp.float32),
        pltpu.VMEM((K, D), jnp.float32),
        pltpu.VMEM((SROWS, D), jnp.float32),
        pltpu.VMEM_SHARED((NPAD, D), jnp.float32),
        pltpu.SemaphoreType.DMA,
        pltpu.SemaphoreType.DMA,
    ],
)(_scat_body)


def _pre_body(dp_ref, x_ref, xp_ref, dinv_ref):
    deg = dp_ref[0:NPAD, 0:1] + dp_ref[NPAD:2 * NPAD, 0:1]
    pos = deg > 0.0
    dinv = jnp.where(pos, lax.rsqrt(jnp.where(pos, deg, 1.0)), 0.0)
    dinv_ref[...] = dinv
    xp_ref[...] = x_ref[...] * dinv


_pre_call = pl.pallas_call(
    _pre_body,
    out_shape=(jax.ShapeDtypeStruct((NPAD, D), jnp.float32),
               jax.ShapeDtypeStruct((NPAD, 1), jnp.float32)),
)


def _post_body(acc_ref, dinv_ref, o_ref):
    o_ref[...] = (acc_ref[0:N, :] + acc_ref[NPAD:NPAD + N, :]) * dinv_ref[0:N]


_post_call = pl.pallas_call(
    _post_body,
    out_shape=jax.ShapeDtypeStruct((N, D), jnp.float32),
)


def kernel(x, edge_index):
    src = edge_index[0].astype(jnp.int32)
    dst = edge_index[1].astype(jnp.int32)
    padidx = jnp.full((EPAD - E,), N, jnp.int32)
    src_p = jnp.concatenate([src, padidx])
    dst_p = jnp.concatenate([dst, padidx])
    x_pad = jnp.pad(x, ((0, NPAD - N), (0, 0)))

    ones_d = jnp.ones((K, D), jnp.float32)
    zeros_d = jnp.zeros((SROWS, D), jnp.float32)

    dp = _deg_call(dst_p, ones_d, zeros_d)
    xp, dinv = _pre_call(dp, x_pad)
    acc = _scat_call(xp, src_p, dst_p, zeros_d)
    return _post_call(acc, dinv)
